# baseline (speedup 1.0000x reference)
import functools

import jax
import jax.numpy as jnp
from jax import lax
from jax.experimental import pallas as pl
from jax.experimental.pallas import tpu as pltpu
from jax.experimental.pallas import tpu_sc as plsc

OUT_FEATURES = 4096
IN_FEATURES = 4096
NUM_CORES = 2
NUM_SUBCORES = 16
NW = NUM_CORES * NUM_SUBCORES
ROWS_PER_W = OUT_FEATURES // NW
CROWS = 8
CCOLS = 1024
CHUNK = CROWS * CCOLS
COL_SLABS = IN_FEATURES // CCOLS
NCHUNK = (ROWS_PER_W // CROWS) * COL_SLABS
NC2 = NCHUNK // 2
LANES = 16
UNROLL = 8

_mesh = plsc.VectorSubcoreMesh(core_axis_name="c", subcore_axis_name="s")


@functools.partial(
    pl.kernel,
    mesh=_mesh,
    out_type=jax.ShapeDtypeStruct((OUT_FEATURES, IN_FEATURES), jnp.float32),
    compiler_params=pltpu.CompilerParams(use_tc_tiling_on_sc=True),
    scratch_types=[
        pltpu.VMEM((CROWS, CCOLS), jnp.int32),
        pltpu.VMEM((CROWS, CCOLS), jnp.int32),
        pltpu.VMEM((CHUNK,), jnp.int32),
        pltpu.VMEM((CHUNK,), jnp.int32),
        pltpu.VMEM((CHUNK,), jnp.float32),
        pltpu.VMEM((CHUNK,), jnp.float32),
        pltpu.VMEM((CROWS, CCOLS), jnp.float32),
        pltpu.VMEM((CROWS, CCOLS), jnp.float32),
        pltpu.VMEM((CROWS, CCOLS), jnp.float32),
        pltpu.VMEM((CROWS, CCOLS), jnp.float32),
        pltpu.SemaphoreType.DMA,
        pltpu.SemaphoreType.DMA,
        pltpu.SemaphoreType.DMA,
        pltpu.SemaphoreType.DMA,
        pltpu.SemaphoreType.DMA,
        pltpu.SemaphoreType.DMA,
        pltpu.SemaphoreType.DMA,
        pltpu.SemaphoreType.DMA,
    ],
)
def _ssl_gather(w_hbm, idx_hbm, g_hbm, out_hbm,
                ib0, ib1, il0, il1, wv0, wv1, gb0, gb1, ob0, ob1,
                si0, si1, sw0, sw1, sg0, sg1, so0, so1):
    wid = lax.axis_index("s") * NUM_CORES + lax.axis_index("c")
    base_row = wid * ROWS_PER_W

    def relayout(ib, il):
        for r in range(CROWS):
            def body(j, _):
                for u in range(UNROLL):
                    c = pl.ds(
                        pl.multiple_of((j * UNROLL + u) * LANES, LANES),
                        LANES)
                    p = pl.ds(
                        pl.multiple_of(r * CCOLS + (j * UNROLL + u) * LANES,
                                       LANES), LANES)
                    il[p] = ib[r, c]
                return 0

            lax.fori_loop(0, CCOLS // (LANES * UNROLL), body, 0)

    def multiply(wv, gb, ob):
        for r in range(CROWS):
            def body(j, _):
                for u in range(UNROLL):
                    c = pl.ds(
                        pl.multiple_of((j * UNROLL + u) * LANES, LANES),
                        LANES)
                    p = pl.ds(
                        pl.multiple_of(r * CCOLS + (j * UNROLL + u) * LANES,
                                       LANES), LANES)
                    ob[r, c] = wv[p] * gb[r, c]
                return 0

            lax.fori_loop(0, CCOLS // (LANES * UNROLL), body, 0)

    def chunk_slice(ref, i):
        row = base_row + (i // COL_SLABS) * CROWS
        col = (i % COL_SLABS) * CCOLS
        return ref.at[pl.ds(row, CROWS), pl.ds(col, CCOLS)]

    def fire_gather(il, wv, sw):
        h = CHUNK // 2
        pltpu.async_copy(w_hbm.at[il.at[pl.ds(0, h)]],
                         wv.at[pl.ds(0, h)], sw)
        pltpu.async_copy(w_hbm.at[il.at[pl.ds(h, h)]],
                         wv.at[pl.ds(h, h)], sw)

    pltpu.sync_copy(chunk_slice(idx_hbm, 0), ib0)
    relayout(ib0, il0)
    fire_gather(il0, wv0, sw0)
    pltpu.async_copy(chunk_slice(idx_hbm, 1), ib1, si1)
    pltpu.async_copy(chunk_slice(g_hbm, 0), gb0, sg0)
    pltpu.async_copy(chunk_slice(g_hbm, 1), gb1, sg1)

    def half(i, i2, iba, ibb, ila, ilb, wva, wvb, gba, oba,
             sia, sib, swa, swb, sga, soa, first):
        last_pair = i2 == NC2 - 1

        def fire_next():
            pltpu.make_async_copy(chunk_slice(idx_hbm, i + 1), ibb,
                                  sib).wait()
            relayout(ibb, ilb)
            fire_gather(ilb, wvb, swb)

        if first:
            fire_next()
        else:
            pl.when(i2 < NC2 - 1)(fire_next)

        @pl.when(i2 < NC2 - 1)
        def _():
            pltpu.async_copy(chunk_slice(idx_hbm, i + 2), iba, sia)

        pltpu.make_async_copy(w_hbm.at[ila], wva, swa).wait()
        pltpu.make_async_copy(chunk_slice(g_hbm, i), gba, sga).wait()
        if first:
            @pl.when(i2 > 0)
            def _():
                pltpu.make_async_copy(oba, chunk_slice(out_hbm, i - 2),
                                      soa).wait()
        else:
            @pl.when(i2 > 0)
            def _():
                pltpu.make_async_copy(oba, chunk_slice(out_hbm, i - 2),
                                      soa).wait()

        multiply(wva, gba, oba)
        pltpu.async_copy(oba, chunk_slice(out_hbm, i), soa)

        @pl.when(i2 < NC2 - 1)
        def _():
            pltpu.async_copy(chunk_slice(g_hbm, i + 2), gba, sga)

    def pair_body(i2, _):
        i = i2 * 2
        half(i, i2, ib0, ib1, il0, il1, wv0, wv1, gb0, ob0,
             si0, si1, sw0, sw1, sg0, so0, first=True)
        half(i + 1, i2, ib1, ib0, il1, il0, wv1, wv0, gb1, ob1,
             si1, si0, sw1, sw0, sg1, so1, first=False)
        return 0

    lax.fori_loop(0, NC2, pair_body, 0)

    pltpu.make_async_copy(ob0, chunk_slice(out_hbm, NCHUNK - 2), so0).wait()
    pltpu.make_async_copy(ob1, chunk_slice(out_hbm, NCHUNK - 1), so1).wait()


def kernel(weight, IDX, G):
    return _ssl_gather(weight, IDX, G)


# speedup vs baseline: 1.0026x; 1.0026x over previous
import functools

import jax
import jax.numpy as jnp
from jax import lax
from jax.experimental import pallas as pl
from jax.experimental.pallas import tpu as pltpu
from jax.experimental.pallas import tpu_sc as plsc

OUT_FEATURES = 4096
IN_FEATURES = 4096
NUM_CORES = 2
NUM_SUBCORES = 16
NW = NUM_CORES * NUM_SUBCORES
ROWS_PER_W = OUT_FEATURES // NW
CROWS = 8
CCOLS = 1024
CHUNK = CROWS * CCOLS
COL_SLABS = IN_FEATURES // CCOLS
NCHUNK = (ROWS_PER_W // CROWS) * COL_SLABS
NC2 = NCHUNK // 2
LANES = 16
UNROLL = 8

_mesh = plsc.VectorSubcoreMesh(core_axis_name="c", subcore_axis_name="s")


@functools.partial(
    pl.kernel,
    mesh=_mesh,
    out_type=jax.ShapeDtypeStruct((OUT_FEATURES, IN_FEATURES), jnp.float32),
    compiler_params=pltpu.CompilerParams(use_tc_tiling_on_sc=True),
    scratch_types=[
        pltpu.VMEM((CROWS, CCOLS), jnp.int32),
        pltpu.VMEM((CROWS, CCOLS), jnp.int32),
        pltpu.VMEM((CHUNK,), jnp.int32),
        pltpu.VMEM((CHUNK,), jnp.int32),
        pltpu.VMEM((CHUNK,), jnp.float32),
        pltpu.VMEM((CHUNK,), jnp.float32),
        pltpu.VMEM((CROWS, CCOLS), jnp.float32),
        pltpu.VMEM((CROWS, CCOLS), jnp.float32),
        pltpu.VMEM((CROWS, CCOLS), jnp.float32),
        pltpu.VMEM((CROWS, CCOLS), jnp.float32),
        pltpu.SemaphoreType.DMA,
        pltpu.SemaphoreType.DMA,
        pltpu.SemaphoreType.DMA,
        pltpu.SemaphoreType.DMA,
        pltpu.SemaphoreType.DMA,
        pltpu.SemaphoreType.DMA,
        pltpu.SemaphoreType.DMA,
        pltpu.SemaphoreType.DMA,
    ],
)
def _ssl_gather(w_hbm, idx_hbm, g_hbm, out_hbm,
                ib0, ib1, il0, il1, wv0, wv1, gb0, gb1, ob0, ob1,
                si0, si1, sw0, sw1, sg0, sg1, so0, so1):
    wid = lax.axis_index("s") * NUM_CORES + lax.axis_index("c")
    base_row = wid * ROWS_PER_W

    def relayout(ib, il):
        for r in range(CROWS):
            def body(j, _):
                for u in range(UNROLL):
                    c = pl.ds(
                        pl.multiple_of((j * UNROLL + u) * LANES, LANES),
                        LANES)
                    p = pl.ds(
                        pl.multiple_of(r * CCOLS + (j * UNROLL + u) * LANES,
                                       LANES), LANES)
                    il[p] = ib[r, c]
                return 0

            lax.fori_loop(0, CCOLS // (LANES * UNROLL), body, 0)

    def multiply(wv, gb, ob):
        for r in range(CROWS):
            def body(j, _):
                for u in range(UNROLL):
                    c = pl.ds(
                        pl.multiple_of((j * UNROLL + u) * LANES, LANES),
                        LANES)
                    p = pl.ds(
                        pl.multiple_of(r * CCOLS + (j * UNROLL + u) * LANES,
                                       LANES), LANES)
                    ob[r, c] = wv[p] * gb[r, c]
                return 0

            lax.fori_loop(0, CCOLS // (LANES * UNROLL), body, 0)

    def chunk_slice(ref, i):
        row = base_row + (i // COL_SLABS) * CROWS
        col = (i % COL_SLABS) * CCOLS
        return ref.at[pl.ds(row, CROWS), pl.ds(col, CCOLS)]

    pltpu.sync_copy(chunk_slice(idx_hbm, 0), ib0)
    relayout(ib0, il0)
    pltpu.async_copy(w_hbm.at[il0], wv0, sw0)
    pltpu.async_copy(chunk_slice(idx_hbm, 1), ib1, si1)
    pltpu.async_copy(chunk_slice(g_hbm, 0), gb0, sg0)
    pltpu.async_copy(chunk_slice(g_hbm, 1), gb1, sg1)

    def half(i, i2, iba, ibb, ila, ilb, wva, wvb, gba, oba,
             sia, sib, swa, swb, sga, soa, first):
        last_pair = i2 == NC2 - 1

        def fire_next():
            pltpu.make_async_copy(chunk_slice(idx_hbm, i + 1), ibb,
                                  sib).wait()
            relayout(ibb, ilb)
            pltpu.async_copy(w_hbm.at[ilb], wvb, swb)

        if first:
            fire_next()
        else:
            pl.when(i2 < NC2 - 1)(fire_next)

        @pl.when(i2 < NC2 - 1)
        def _():
            pltpu.async_copy(chunk_slice(idx_hbm, i + 2), iba, sia)

        pltpu.make_async_copy(w_hbm.at[ila], wva, swa).wait()
        pltpu.make_async_copy(chunk_slice(g_hbm, i), gba, sga).wait()
        if first:
            @pl.when(i2 > 0)
            def _():
                pltpu.make_async_copy(oba, chunk_slice(out_hbm, i - 2),
                                      soa).wait()
        else:
            @pl.when(i2 > 0)
            def _():
                pltpu.make_async_copy(oba, chunk_slice(out_hbm, i - 2),
                                      soa).wait()

        multiply(wva, gba, oba)
        pltpu.async_copy(oba, chunk_slice(out_hbm, i), soa)

        @pl.when(i2 < NC2 - 1)
        def _():
            pltpu.async_copy(chunk_slice(g_hbm, i + 2), gba, sga)

    def pair_body(i2, _):
        i = i2 * 2
        half(i, i2, ib0, ib1, il0, il1, wv0, wv1, gb0, ob0,
             si0, si1, sw0, sw1, sg0, so0, first=True)
        half(i + 1, i2, ib1, ib0, il1, il0, wv1, wv0, gb1, ob1,
             si1, si0, sw1, sw0, sg1, so1, first=False)
        return 0

    lax.fori_loop(0, NC2, pair_body, 0)

    pltpu.make_async_copy(ob0, chunk_slice(out_hbm, NCHUNK - 2), so0).wait()
    pltpu.make_async_copy(ob1, chunk_slice(out_hbm, NCHUNK - 1), so1).wait()


def kernel(weight, IDX, G):
    return _ssl_gather(weight, IDX, G)
